# Initial kernel scaffold; baseline (speedup 1.0000x reference)
import jax
import jax.numpy as jnp
from jax.experimental import pallas as pl


def kernel(x, gate_W, gate_b, W1, b1, W2, b2):
    raise NotImplementedError("write your pallas kernel here")


# speedup vs baseline: 1.2696x; 1.2696x over previous
import jax
import jax.numpy as jnp
from jax import lax
from jax.experimental import pallas as pl
from jax.experimental.pallas import tpu as pltpu


def _router_body(x_ref, gw_ref, gb_ref, wd_ref, psum_ref, loss_ref, *, n_tokens, n_experts):
    t = pl.program_id(0)
    xs = x_ref[...]
    gw = gw_ref[...]
    logits = lax.dot_general(xs, gw, (((1,), (1,)), ((), ())),
                             preferred_element_type=jnp.float32) + gb_ref[...]
    m = jnp.max(logits, axis=1, keepdims=True)
    p = jnp.exp(logits - m)
    probs = p / jnp.sum(p, axis=1, keepdims=True)
    iota = lax.broadcasted_iota(jnp.int32, probs.shape, 1)
    m1 = jnp.max(probs, axis=1, keepdims=True)
    i0 = jnp.min(jnp.where(probs == m1, iota, n_experts), axis=1, keepdims=True)
    probs2 = jnp.where(iota == i0, -1.0, probs)
    m2 = jnp.max(probs2, axis=1, keepdims=True)
    i1 = jnp.min(jnp.where(probs2 == m2, iota, n_experts), axis=1, keepdims=True)
    wd_ref[...] = jnp.where(iota == i0, m1, 0.0) + jnp.where(iota == i1, m2, 0.0)

    @pl.when(t == 0)
    def _():
        psum_ref[...] = jnp.zeros_like(psum_ref)

    psum_ref[...] += jnp.sum(probs, axis=0, keepdims=True)

    @pl.when(t == pl.num_programs(0) - 1)
    def _():
        mean = psum_ref[...] * (1.0 / n_tokens)
        loss_ref[...] = jnp.sum(mean * mean, axis=1, keepdims=True) * n_experts


def _ffn_body(x_ref, wd_ref, w1_ref, b1_ref, w2_ref, b2_ref, out_ref, *, chunk, n_experts):
    e = pl.program_id(0)
    f = pl.program_id(1)

    @pl.when((e == 0) & (f == 0))
    def _():
        out_ref[...] = jnp.zeros_like(out_ref)

    w1 = w1_ref[0]
    w2 = w2_ref[0]
    b1 = b1_ref[0]
    b2 = b2_ref[0]
    bias_scale = (f == 0).astype(jnp.float32)
    onehot = (lax.broadcasted_iota(jnp.int32, (n_experts, 1), 0) == e).astype(jnp.float32)
    n_tokens = x_ref.shape[0]
    for c in range(n_tokens // chunk):
        xs = x_ref[pl.ds(c * chunk, chunk), :]
        wtile = wd_ref[pl.ds(c * chunk, chunk), :]
        w = lax.dot_general(wtile, onehot, (((1,), (0,)), ((), ())),
                            preferred_element_type=jnp.float32)
        h = lax.dot_general(xs, w1, (((1,), (1,)), ((), ())),
                            preferred_element_type=jnp.float32)
        h = jnp.maximum(h + b1, 0.0)
        proc = lax.dot_general(h, w2, (((1,), (1,)), ((), ())),
                               preferred_element_type=jnp.float32)
        out_ref[pl.ds(c * chunk, chunk), :] += (proc + b2 * bias_scale) * w


def kernel(x, gate_W, gate_b, W1, b1, W2, b2):
    seq_len, batch, d = x.shape
    n_experts, dff, _ = W1.shape
    tokens = seq_len * batch
    x_flat = x.reshape(tokens, d)

    rt = 512 if tokens % 512 == 0 else tokens
    import functools
    wd, _, loss = pl.pallas_call(
        functools.partial(_router_body, n_tokens=tokens, n_experts=n_experts),
        grid=(tokens // rt,),
        in_specs=[
            pl.BlockSpec((rt, d), lambda t: (t, 0)),
            pl.BlockSpec((n_experts, d), lambda t: (0, 0)),
            pl.BlockSpec((1, n_experts), lambda t: (0, 0)),
        ],
        out_specs=[
            pl.BlockSpec((rt, n_experts), lambda t: (t, 0)),
            pl.BlockSpec((1, n_experts), lambda t: (0, 0)),
            pl.BlockSpec((1, 1), lambda t: (0, 0)),
        ],
        out_shape=[
            jax.ShapeDtypeStruct((tokens, n_experts), jnp.float32),
            jax.ShapeDtypeStruct((1, n_experts), jnp.float32),
            jax.ShapeDtypeStruct((1, 1), jnp.float32),
        ],
    )(x_flat, gate_W, gate_b.reshape(1, n_experts))

    fsplit = 4
    dff_blk = dff // fsplit
    chunk = 256 if tokens % 256 == 0 else tokens
    out_flat = pl.pallas_call(
        functools.partial(_ffn_body, chunk=chunk, n_experts=n_experts),
        grid=(n_experts, fsplit),
        in_specs=[
            pl.BlockSpec((tokens, d), lambda e, f: (0, 0)),
            pl.BlockSpec((tokens, n_experts), lambda e, f: (0, 0)),
            pl.BlockSpec((1, dff_blk, d), lambda e, f: (e, f, 0)),
            pl.BlockSpec((1, 1, dff_blk), lambda e, f: (e, 0, f)),
            pl.BlockSpec((1, d, dff_blk), lambda e, f: (e, 0, f)),
            pl.BlockSpec((1, 1, d), lambda e, f: (e, 0, 0)),
        ],
        out_specs=pl.BlockSpec((tokens, d), lambda e, f: (0, 0)),
        out_shape=jax.ShapeDtypeStruct((tokens, d), jnp.float32),
    )(x_flat, wd, W1, b1.reshape(n_experts, 1, dff), W2, b2.reshape(n_experts, 1, d))

    return (out_flat.reshape(seq_len, batch, d), loss.reshape(()))
